# Initial kernel scaffold; baseline (speedup 1.0000x reference)
import jax
import jax.numpy as jnp
from jax.experimental import pallas as pl


def kernel(embeds, edge_index, edge_weight):
    raise NotImplementedError("write your pallas kernel here")


# speedup vs baseline: 4.4791x; 4.4791x over previous
import functools

import jax
import jax.numpy as jnp
from jax import lax
from jax.experimental import pallas as pl
from jax.experimental.pallas import tpu as pltpu
from jax.experimental.pallas import tpu_sc as plsc

N = 10000
E = 320000
D = 128
NUM_LAYERS = 4

NC = 2
NS = 16
NW = NC * NS
C = 128
NCHUNK = E // C
RPT = 624
REM = N - RPT * NS
ZR = 208
DB = D // 16

_mesh = plsc.VectorSubcoreMesh(
    core_axis_name="c", subcore_axis_name="s", num_cores=NC, num_subcores=NS
)


def _prop_body(x_hbm, col_hbm, row_hbm, w_hbm, out_hbm,
               colv, rowv, wv, rowsv, zerov, acc, sem):
    cid = lax.axis_index("c")
    sid = lax.axis_index("s")
    wid = sid * NC + cid

    zvec = jnp.zeros((16,), jnp.float32)

    def zfill(i, carry):
        for dblk in range(DB):
            zerov[i, pl.ds(dblk * 16, 16)] = zvec
        return carry

    lax.fori_loop(0, ZR, zfill, None)
    for k in range(RPT // ZR):
        pltpu.sync_copy(zerov, acc.at[pl.ds(sid * RPT + k * ZR, ZR)])

    @pl.when(sid == 0)
    def _():
        pltpu.sync_copy(zerov.at[pl.ds(0, REM)], acc.at[pl.ds(RPT * NS, REM)])

    plsc.subcore_barrier()

    nit = (NCHUNK - wid + (NW - 1)) // NW

    def it_body(it, carry):
        base = (wid + it * NW) * C
        pltpu.sync_copy(col_hbm.at[pl.ds(base, C)], colv)
        pltpu.sync_copy(row_hbm.at[pl.ds(base, C)], rowv)
        pltpu.sync_copy(w_hbm.at[pl.ds(base, C)], wv)
        pltpu.async_copy(x_hbm.at[colv], rowsv, sem).wait()

        def scale(g, c2):
            wvec = wv[pl.ds(g * 16, 16)]
            for lane in range(16):
                we = wvec[lane]
                e = g * 16 + lane
                for dblk in range(DB):
                    sl = pl.ds(dblk * 16, 16)
                    rowsv[e, sl] = rowsv[e, sl] * we
            return c2

        lax.fori_loop(0, C // 16, scale, None)
        pltpu.sync_copy(rowsv, acc.at[rowv], add=True)
        return carry

    lax.fori_loop(0, nit, it_body, None)

    plsc.subcore_barrier()
    r0 = sid * RPT
    pltpu.sync_copy(acc.at[pl.ds(r0, RPT)],
                    out_hbm.at[pl.ds(cid * N + r0, RPT)])

    @pl.when(sid == 0)
    def _():
        pltpu.sync_copy(acc.at[pl.ds(RPT * NS, REM)],
                        out_hbm.at[pl.ds(cid * N + RPT * NS, REM)])


_sc_propagate = pl.kernel(
    _prop_body,
    out_type=jax.ShapeDtypeStruct((NC * N, D), jnp.float32),
    mesh=_mesh,
    scratch_types=[
        pltpu.VMEM((C,), jnp.int32),
        pltpu.VMEM((C,), jnp.int32),
        pltpu.VMEM((C,), jnp.float32),
        pltpu.VMEM((C, D), jnp.float32),
        pltpu.VMEM((ZR, D), jnp.float32),
        pltpu.VMEM_SHARED((N, D), jnp.float32),
        pltpu.SemaphoreType.DMA,
    ],
)


def _combine_body(p0_ref, p1_ref, acc_ref, x_ref, accn_ref, *, scale):
    x = p0_ref[...] + p1_ref[...]
    x_ref[...] = x
    accn_ref[...] = (acc_ref[...] + x) * scale


def _combine(p0, p1, acc, scale):
    bn = 400
    grid = N // bn
    bs = pl.BlockSpec((bn, D), lambda i: (i, 0))
    return pl.pallas_call(
        functools.partial(_combine_body, scale=scale),
        grid=(grid,),
        in_specs=[bs, bs, bs],
        out_specs=[bs, bs],
        out_shape=[
            jax.ShapeDtypeStruct((N, D), jnp.float32),
            jax.ShapeDtypeStruct((N, D), jnp.float32),
        ],
    )(p0, p1, acc)


def kernel(embeds, edge_index, edge_weight):
    row = edge_index[0]
    col = edge_index[1]
    x = embeds
    acc = embeds
    for layer in range(NUM_LAYERS):
        p = _sc_propagate(x, col, row, edge_weight)
        scale = 1.0 if layer < NUM_LAYERS - 1 else 1.0 / (NUM_LAYERS + 1)
        x, acc = _combine(p[:N], p[N:], acc, scale)
    return acc
